# baseline (speedup 1.0000x reference)
import jax
import jax.numpy as jnp
from jax.experimental import pallas as pl
from jax.experimental.pallas import tpu as pltpu
from jax.experimental.pallas import tpu_sc as plsc

ROWS = 16
HIDDEN = 4096


def _body(table_hbm, idx_hbm, out_hbm):
    pltpu.sync_copy(table_hbm, out_hbm)


def kernel(table, seq_indices):
    mesh = plsc.ScalarSubcoreMesh(axis_name="c", num_cores=1)
    out = pl.kernel(
        _body,
        mesh=mesh,
        out_type=jax.ShapeDtypeStruct((ROWS, HIDDEN), jnp.float32),
    )(table, seq_indices)
    return out[None]


# speedup vs baseline: 1.2561x; 1.2561x over previous
import jax
import jax.numpy as jnp
from jax import lax
from jax.experimental import pallas as pl
from jax.experimental.pallas import tpu as pltpu
from jax.experimental.pallas import tpu_sc as plsc

ROWS = 16
HIDDEN = 4096
NC = 1
NS = 16
NW = NC * NS
CHUNK = HIDDEN // NW


def _body(table_hbm, idx_hbm, out_hbm, idx_v, rows_v, sem):
    c = lax.axis_index("c")
    s = lax.axis_index("s")
    wid = s * NC + c
    col = wid * CHUNK

    pltpu.sync_copy(idx_hbm, idx_v)
    idx = idx_v[...]

    pltpu.async_copy(table_hbm.at[idx, pl.ds(col, CHUNK)], rows_v, sem).wait()
    pltpu.sync_copy(rows_v, out_hbm.at[:, pl.ds(col, CHUNK)])


def kernel(table, seq_indices):
    mesh = plsc.VectorSubcoreMesh(
        core_axis_name="c", subcore_axis_name="s", num_cores=1
    )
    out = pl.kernel(
        _body,
        mesh=mesh,
        out_type=jax.ShapeDtypeStruct((ROWS, HIDDEN), jnp.float32),
        scratch_types=[
            pltpu.VMEM((ROWS,), jnp.int32),
            pltpu.VMEM((ROWS, CHUNK), jnp.float32),
            pltpu.SemaphoreType.DMA,
        ],
    )(table, seq_indices)
    return out[None]


# speedup vs baseline: 1.2705x; 1.0115x over previous
import jax
import jax.numpy as jnp
from jax import lax
from jax.experimental import pallas as pl
from jax.experimental.pallas import tpu as pltpu
from jax.experimental.pallas import tpu_sc as plsc

ROWS = 16
HIDDEN = 4096
NC = 1
NS = 16
NW = NC * NS
CHUNK = HIDDEN // NW


def _body(table_hbm, idx_hbm, out_hbm, idx_v, row_v, sem):
    c = lax.axis_index("c")
    s = lax.axis_index("s")
    wid = s * NC + c

    pltpu.sync_copy(idx_hbm, idx_v)
    dn = lax.GatherDimensionNumbers(
        offset_dims=(), collapsed_slice_dims=(0,), start_index_map=(0,)
    )
    widvec = jnp.full((ROWS,), wid, jnp.int32)
    idx_v[...] = lax.gather(
        idx_v[...], widvec[:, None], dn, slice_sizes=(1,),
        mode=lax.GatherScatterMode.PROMISE_IN_BOUNDS,
    )

    pltpu.async_copy(
        table_hbm.at[idx_v.at[pl.ds(0, 1)]], row_v, sem
    ).wait()
    pltpu.sync_copy(row_v, out_hbm.at[pl.ds(wid, 1), :])


def kernel(table, seq_indices):
    mesh = plsc.VectorSubcoreMesh(
        core_axis_name="c", subcore_axis_name="s", num_cores=1
    )
    out = pl.kernel(
        _body,
        mesh=mesh,
        out_type=jax.ShapeDtypeStruct((ROWS, HIDDEN), jnp.float32),
        scratch_types=[
            pltpu.VMEM((ROWS,), jnp.int32),
            pltpu.VMEM((1, HIDDEN), jnp.float32),
            pltpu.SemaphoreType.DMA,
        ],
    )(table, seq_indices)
    return out[None]
